# Initial kernel scaffold; baseline (speedup 1.0000x reference)
import jax
import jax.numpy as jnp
from jax.experimental import pallas as pl


def kernel(x, edge_index, W1, b1, W2, b2, W3, b3):
    raise NotImplementedError("write your pallas kernel here")


# speedup vs baseline: 6.2963x; 6.2963x over previous
import functools

import jax
import jax.numpy as jnp
from jax import lax
from jax.experimental import pallas as pl
from jax.experimental.pallas import tpu as pltpu
from jax.experimental.pallas import tpu_sc as plsc

N = 10000
N_PAD = 10240
E = 160000
NT = 16
ROWS_PER_TILE = N_PAD // NT

EDGES_PER_TILE = E // NT
CHUNK = 80
N_CHUNKS = EDGES_PER_TILE // CHUNK

DEG_EPW = E // 32
DEG_CHUNK = 40
DEG_W = 128
DEG_CHUNKS = DEG_EPW // DEG_CHUNK
ZROWS = 128


def _sc_mesh():
    return plsc.VectorSubcoreMesh(core_axis_name="c", subcore_axis_name="s")


def _degree_counts(dst, ones_rows, zero_rows):
    @functools.partial(
        pl.kernel,
        out_type=(jax.ShapeDtypeStruct((N_PAD, DEG_W), jnp.float32),
                  jax.ShapeDtypeStruct((N_PAD, DEG_W), jnp.float32)),
        mesh=_sc_mesh(),
        scratch_types=[
            pltpu.VMEM((1, DEG_CHUNK), jnp.int32),
            pltpu.VMEM((DEG_CHUNK, DEG_W), jnp.float32),
            pltpu.VMEM_SHARED((N_PAD, DEG_W), jnp.float32),
        ],
    )
    def k(dst_h, ones_h, zeros_h, cnt0_h, cnt1_h, idx_v, ones_v, cnt_sh):
        c = lax.axis_index("c")
        t = lax.axis_index("s")
        pltpu.sync_copy(ones_h, ones_v)
        for j in range(ROWS_PER_TILE // ZROWS):
            pltpu.sync_copy(zeros_h,
                            cnt_sh.at[pl.ds(t * ROWS_PER_TILE + j * ZROWS, ZROWS)])
        plsc.subcore_barrier()
        base = c * (E // 2) + t * DEG_EPW

        @pl.loop(0, DEG_CHUNKS)
        def _(kk):
            off = base + kk * DEG_CHUNK
            pltpu.sync_copy(dst_h.at[pl.ds(off, DEG_CHUNK)], idx_v.at[0])
            pltpu.sync_copy(ones_v, cnt_sh.at[idx_v.at[0]], add=True)

        plsc.subcore_barrier()
        rs = pl.ds(t * ROWS_PER_TILE, ROWS_PER_TILE)

        @pl.when(c == 0)
        def _():
            pltpu.sync_copy(cnt_sh.at[rs], cnt0_h.at[rs])

        @pl.when(c == 1)
        def _():
            pltpu.sync_copy(cnt_sh.at[rs], cnt1_h.at[rs])

    return k(dst, ones_rows, zero_rows)


def _propagate(h_left, h_right, src, dst, dh):
    @functools.partial(
        pl.kernel,
        out_type=(jax.ShapeDtypeStruct((N_PAD, dh), jnp.float32),
                  jax.ShapeDtypeStruct((N_PAD, dh), jnp.float32)),
        mesh=_sc_mesh(),
        scratch_types=[
            pltpu.VMEM((2, CHUNK), jnp.int32),
            pltpu.VMEM((CHUNK, dh), jnp.float32),
            pltpu.VMEM_SHARED((N_PAD, dh), jnp.float32),
            pltpu.SemaphoreType.DMA,
        ],
    )
    def k(hl_h, hr_h, src_h, dst_h, outl_h, outr_h, idx_v, rows_v, acc_sh, sem):
        c = lax.axis_index("c")
        t = lax.axis_index("s")
        rs = pl.ds(t * ROWS_PER_TILE, ROWS_PER_TILE)

        def work(h_h, out_h):
            pltpu.sync_copy(h_h.at[rs], acc_sh.at[rs])
            plsc.subcore_barrier()
            base = t * EDGES_PER_TILE

            @pl.loop(0, N_CHUNKS)
            def _(kk):
                off = base + kk * CHUNK
                pltpu.sync_copy(src_h.at[pl.ds(off, CHUNK)], idx_v.at[0])
                pltpu.sync_copy(dst_h.at[pl.ds(off, CHUNK)], idx_v.at[1])
                pltpu.async_copy(h_h.at[idx_v.at[0]], rows_v, sem).wait()
                pltpu.sync_copy(rows_v, acc_sh.at[idx_v.at[1]], add=True)

            plsc.subcore_barrier()
            pltpu.sync_copy(acc_sh.at[rs], out_h.at[rs])

        @pl.when(c == 0)
        def _():
            work(hl_h, outl_h)

        @pl.when(c == 1)
        def _():
            work(hr_h, outr_h)

    return k(h_left, h_right, src, dst)


BLK = 1024


def _dinv_block(c0_ref, c1_ref):
    deg = c0_ref[:, 0:1] + c1_ref[:, 0:1] + 1.0
    return lax.rsqrt(deg)


def _mm_first(x, w, cnt0, cnt1):
    d_out = w.shape[1]
    dh = d_out // 2

    def body(x_ref, w_ref, c0_ref, c1_ref, ol_ref, or_ref):
        dinv = _dinv_block(c0_ref, c1_ref)
        h = jnp.dot(x_ref[...], w_ref[...],
                    preferred_element_type=jnp.float32) * dinv
        ol_ref[...] = h[:, :dh]
        or_ref[...] = h[:, dh:]

    return pl.pallas_call(
        body,
        grid=(N_PAD // BLK,),
        in_specs=[
            pl.BlockSpec((BLK, x.shape[1]), lambda i: (i, 0)),
            pl.BlockSpec(w.shape, lambda i: (0, 0)),
            pl.BlockSpec((BLK, DEG_W), lambda i: (i, 0)),
            pl.BlockSpec((BLK, DEG_W), lambda i: (i, 0)),
        ],
        out_specs=[
            pl.BlockSpec((BLK, dh), lambda i: (i, 0)),
            pl.BlockSpec((BLK, dh), lambda i: (i, 0)),
        ],
        out_shape=[
            jax.ShapeDtypeStruct((N_PAD, dh), jnp.float32),
            jax.ShapeDtypeStruct((N_PAD, dh), jnp.float32),
        ],
    )(x, w, cnt0, cnt1)


def _mm_mid(accl, accr, cnt0, cnt1, b_row, w):
    d_in = accl.shape[1] + accr.shape[1]
    dh_in = accl.shape[1]
    d_out = w.shape[1]
    dh = d_out // 2

    def body(al_ref, ar_ref, c0_ref, c1_ref, b_ref, w_ref, ol_ref, or_ref):
        dinv = _dinv_block(c0_ref, c1_ref)
        a = jnp.concatenate([al_ref[...], ar_ref[...]], axis=1)
        xr = jnp.maximum(a * dinv + b_ref[...], 0.0)
        h = jnp.dot(xr, w_ref[...], preferred_element_type=jnp.float32) * dinv
        ol_ref[...] = h[:, :dh]
        or_ref[...] = h[:, dh:]

    return pl.pallas_call(
        body,
        grid=(N_PAD // BLK,),
        in_specs=[
            pl.BlockSpec((BLK, dh_in), lambda i: (i, 0)),
            pl.BlockSpec((BLK, dh_in), lambda i: (i, 0)),
            pl.BlockSpec((BLK, DEG_W), lambda i: (i, 0)),
            pl.BlockSpec((BLK, DEG_W), lambda i: (i, 0)),
            pl.BlockSpec((1, d_in), lambda i: (0, 0)),
            pl.BlockSpec(w.shape, lambda i: (0, 0)),
        ],
        out_specs=[
            pl.BlockSpec((BLK, dh), lambda i: (i, 0)),
            pl.BlockSpec((BLK, dh), lambda i: (i, 0)),
        ],
        out_shape=[
            jax.ShapeDtypeStruct((N_PAD, dh), jnp.float32),
            jax.ShapeDtypeStruct((N_PAD, dh), jnp.float32),
        ],
    )(accl, accr, cnt0, cnt1, b_row, w)


def _scale_only(accl, accr, cnt0, cnt1, b_row):
    d_in = accl.shape[1] + accr.shape[1]
    dh_in = accl.shape[1]

    def body(al_ref, ar_ref, c0_ref, c1_ref, b_ref, ol_ref, or_ref):
        dinv = _dinv_block(c0_ref, c1_ref)
        a = jnp.concatenate([al_ref[...], ar_ref[...]], axis=1)
        xr = jnp.maximum(a * dinv + b_ref[...], 0.0) * dinv
        ol_ref[...] = xr[:, :dh_in]
        or_ref[...] = xr[:, dh_in:]

    return pl.pallas_call(
        body,
        grid=(N_PAD // BLK,),
        in_specs=[
            pl.BlockSpec((BLK, dh_in), lambda i: (i, 0)),
            pl.BlockSpec((BLK, dh_in), lambda i: (i, 0)),
            pl.BlockSpec((BLK, DEG_W), lambda i: (i, 0)),
            pl.BlockSpec((BLK, DEG_W), lambda i: (i, 0)),
            pl.BlockSpec((1, d_in), lambda i: (0, 0)),
        ],
        out_specs=[
            pl.BlockSpec((BLK, dh_in), lambda i: (i, 0)),
            pl.BlockSpec((BLK, dh_in), lambda i: (i, 0)),
        ],
        out_shape=[
            jax.ShapeDtypeStruct((N_PAD, dh_in), jnp.float32),
            jax.ShapeDtypeStruct((N_PAD, dh_in), jnp.float32),
        ],
    )(accl, accr, cnt0, cnt1, b_row)


def _final(acc3l, acc3r, cnt0, cnt1, w3, b_row):
    dh_in = acc3l.shape[1]
    n_cls = w3.shape[1]

    def body(al_ref, ar_ref, c0_ref, c1_ref, w_ref, b_ref, o_ref):
        dinv = _dinv_block(c0_ref, c1_ref)
        a = jnp.concatenate([al_ref[...], ar_ref[...]], axis=1) * dinv
        logits = jnp.dot(a, w_ref[...],
                         preferred_element_type=jnp.float32) + b_ref[...]
        m = jnp.max(logits, axis=1, keepdims=True)
        e = jnp.exp(logits - m)
        lse = jnp.log(jnp.sum(e, axis=1, keepdims=True)) + m
        o_ref[...] = logits - lse

    return pl.pallas_call(
        body,
        grid=(N_PAD // BLK,),
        in_specs=[
            pl.BlockSpec((BLK, dh_in), lambda i: (i, 0)),
            pl.BlockSpec((BLK, dh_in), lambda i: (i, 0)),
            pl.BlockSpec((BLK, DEG_W), lambda i: (i, 0)),
            pl.BlockSpec((BLK, DEG_W), lambda i: (i, 0)),
            pl.BlockSpec(w3.shape, lambda i: (0, 0)),
            pl.BlockSpec((1, n_cls), lambda i: (0, 0)),
        ],
        out_specs=pl.BlockSpec((BLK, n_cls), lambda i: (i, 0)),
        out_shape=jax.ShapeDtypeStruct((N_PAD, n_cls), jnp.float32),
    )(acc3l, acc3r, cnt0, cnt1, w3, b_row)


def kernel(x, edge_index, W1, b1, W2, b2, W3, b3):
    src = edge_index[0]
    dst = edge_index[1]
    x_pad = jnp.pad(x, ((0, N_PAD - N), (0, 0)))
    ones_rows = jnp.ones((DEG_CHUNK, DEG_W), jnp.float32)
    zero_rows = jnp.zeros((ZROWS, DEG_W), jnp.float32)

    cnt0, cnt1 = _degree_counts(dst, ones_rows, zero_rows)

    hl1, hr1 = _mm_first(x_pad, W1, cnt0, cnt1)
    acc1l, acc1r = _propagate(hl1, hr1, src, dst, 128)

    hl2, hr2 = _mm_mid(acc1l, acc1r, cnt0, cnt1, b1.reshape(1, -1), W2)
    acc2l, acc2r = _propagate(hl2, hr2, src, dst, 128)

    hl3, hr3 = _scale_only(acc2l, acc2r, cnt0, cnt1, b2.reshape(1, -1))
    acc3l, acc3r = _propagate(hl3, hr3, src, dst, 128)

    out = _final(acc3l, acc3r, cnt0, cnt1, W3, b3.reshape(1, -1))
    return out[:N]


# speedup vs baseline: 6.7127x; 1.0661x over previous
import functools

import jax
import jax.numpy as jnp
from jax import lax
from jax.experimental import pallas as pl
from jax.experimental.pallas import tpu as pltpu
from jax.experimental.pallas import tpu_sc as plsc

N = 10000
N_PAD = 10240
E = 160000
E_PAD = 163840
PAD_ROW = N_PAD - 8
NT = 16
ROWS_PER_TILE = N_PAD // NT

CHUNK = 128
CHT = (E_PAD // NT) // CHUNK
DEG_CHT = (E_PAD // 32) // CHUNK
IDXG = 40
DEG_W = 128
ZROWS = 128


def _sc_mesh():
    return plsc.VectorSubcoreMesh(core_axis_name="c", subcore_axis_name="s")


def _degree_counts(dst3, ones_rows, zero_rows):
    @functools.partial(
        pl.kernel,
        out_type=(jax.ShapeDtypeStruct((N_PAD, DEG_W), jnp.float32),
                  jax.ShapeDtypeStruct((N_PAD, DEG_W), jnp.float32)),
        mesh=_sc_mesh(),
        scratch_types=[
            pltpu.VMEM((DEG_CHT, CHUNK), jnp.int32),
            pltpu.VMEM((CHUNK, DEG_W), jnp.float32),
            pltpu.VMEM_SHARED((N_PAD, DEG_W), jnp.float32),
            pltpu.SemaphoreType.DMA,
        ],
    )
    def k(dst_h, ones_h, zeros_h, cnt0_h, cnt1_h, dstv, ones_v, cnt_sh, sem):
        c = lax.axis_index("c")
        t = lax.axis_index("s")
        w = c * NT + t
        pltpu.sync_copy(dst_h.at[w], dstv)
        pltpu.sync_copy(ones_h, ones_v)
        for j in range(ROWS_PER_TILE // ZROWS):
            pltpu.sync_copy(zeros_h,
                            cnt_sh.at[pl.ds(t * ROWS_PER_TILE + j * ZROWS, ZROWS)])
        plsc.subcore_barrier()

        @pl.loop(0, DEG_CHT)
        def _(kk):
            pltpu.async_copy(ones_v, cnt_sh.at[dstv.at[kk]], sem, add=True)

        @pl.loop(0, DEG_CHT)
        def _(kk):
            pltpu.make_async_copy(ones_v, cnt_sh.at[dstv.at[0]], sem).wait()

        plsc.subcore_barrier()
        rs = pl.ds(t * ROWS_PER_TILE, ROWS_PER_TILE)

        @pl.when(c == 0)
        def _():
            pltpu.sync_copy(cnt_sh.at[rs], cnt0_h.at[rs])

        @pl.when(c == 1)
        def _():
            pltpu.sync_copy(cnt_sh.at[rs], cnt1_h.at[rs])

    return k(dst3, ones_rows, zero_rows)


def _propagate(h_left, h_right, src2, dst2, dh):
    @functools.partial(
        pl.kernel,
        out_type=(jax.ShapeDtypeStruct((N_PAD, dh), jnp.float32),
                  jax.ShapeDtypeStruct((N_PAD, dh), jnp.float32)),
        mesh=_sc_mesh(),
        scratch_types=[
            pltpu.VMEM((IDXG, CHUNK), jnp.int32),
            pltpu.VMEM((IDXG, CHUNK), jnp.int32),
            pltpu.VMEM((CHUNK, dh), jnp.float32),
            pltpu.VMEM((CHUNK, dh), jnp.float32),
            pltpu.VMEM_SHARED((N_PAD, dh), jnp.float32),
            pltpu.SemaphoreType.DMA,
            pltpu.SemaphoreType.DMA,
        ],
    )
    def k(hl_h, hr_h, src_h, dst_h, outl_h, outr_h,
          srcb, dstb, rows_a, rows_b, acc_sh, sem_a, sem_b):
        c = lax.axis_index("c")
        t = lax.axis_index("s")
        rs = pl.ds(t * ROWS_PER_TILE, ROWS_PER_TILE)

        def work(h_h, out_h):
            pltpu.sync_copy(h_h.at[rs], acc_sh.at[rs])
            plsc.subcore_barrier()

            @pl.loop(0, CHT // IDXG)
            def _(g):
                base = t * CHT + g * IDXG
                pltpu.sync_copy(src_h.at[pl.ds(base, IDXG)], srcb)
                pltpu.sync_copy(dst_h.at[pl.ds(base, IDXG)], dstb)
                pltpu.async_copy(h_h.at[srcb.at[0]], rows_a, sem_a)

                @pl.loop(0, IDXG, step=2)
                def _(j):
                    pltpu.make_async_copy(h_h.at[srcb.at[j]], rows_a,
                                          sem_a).wait()
                    pltpu.async_copy(h_h.at[srcb.at[j + 1]], rows_b, sem_b)
                    pltpu.sync_copy(rows_a, acc_sh.at[dstb.at[j]], add=True)
                    pltpu.make_async_copy(h_h.at[srcb.at[j + 1]], rows_b,
                                          sem_b).wait()

                    @pl.when(j + 2 < IDXG)
                    def _():
                        pltpu.async_copy(h_h.at[srcb.at[j + 2]], rows_a, sem_a)

                    pltpu.sync_copy(rows_b, acc_sh.at[dstb.at[j + 1]],
                                    add=True)

            plsc.subcore_barrier()
            pltpu.sync_copy(acc_sh.at[rs], out_h.at[rs])

        @pl.when(c == 0)
        def _():
            work(hl_h, outl_h)

        @pl.when(c == 1)
        def _():
            work(hr_h, outr_h)

    return k(h_left, h_right, src2, dst2)


def _propagate_last(ha, hb, src2, dst2):
    @functools.partial(
        pl.kernel,
        out_type=(jax.ShapeDtypeStruct((N_PAD, 128), jnp.float32),
                  jax.ShapeDtypeStruct((N_PAD, 128), jnp.float32)),
        mesh=_sc_mesh(),
        scratch_types=[
            pltpu.VMEM((DEG_CHT, CHUNK), jnp.int32),
            pltpu.VMEM((DEG_CHT, CHUNK), jnp.int32),
            pltpu.VMEM((CHUNK, 128), jnp.float32),
            pltpu.VMEM((CHUNK, 128), jnp.float32),
            pltpu.VMEM_SHARED((N_PAD, 128), jnp.float32),
            pltpu.SemaphoreType.DMA,
            pltpu.SemaphoreType.DMA,
        ],
    )
    def k(ha_h, hb_h, src_h, dst_h, out0_h, out1_h,
          srcb, dstb, rows_a, rows_b, acc_sh, sem_a, sem_b):
        c = lax.axis_index("c")
        t = lax.axis_index("s")
        w = c * NT + t
        rs = pl.ds(t * ROWS_PER_TILE, ROWS_PER_TILE)

        def work(h_h, out_h):
            pltpu.sync_copy(h_h.at[rs], acc_sh.at[rs])
            plsc.subcore_barrier()

            base = w * DEG_CHT
            pltpu.sync_copy(src_h.at[pl.ds(base, DEG_CHT)], srcb)
            pltpu.sync_copy(dst_h.at[pl.ds(base, DEG_CHT)], dstb)
            pltpu.async_copy(h_h.at[srcb.at[0]], rows_a, sem_a)

            @pl.loop(0, DEG_CHT, step=2)
            def _(j):
                pltpu.make_async_copy(h_h.at[srcb.at[j]], rows_a,
                                      sem_a).wait()
                pltpu.async_copy(h_h.at[srcb.at[j + 1]], rows_b, sem_b)
                pltpu.sync_copy(rows_a, acc_sh.at[dstb.at[j]], add=True)
                pltpu.make_async_copy(h_h.at[srcb.at[j + 1]], rows_b,
                                      sem_b).wait()

                @pl.when(j + 2 < DEG_CHT)
                def _():
                    pltpu.async_copy(h_h.at[srcb.at[j + 2]], rows_a,
                                     sem_a)

                pltpu.sync_copy(rows_b, acc_sh.at[dstb.at[j + 1]],
                                add=True)

            plsc.subcore_barrier()
            pltpu.sync_copy(acc_sh.at[rs], out_h.at[rs])

        @pl.when(c == 0)
        def _():
            work(ha_h, out0_h)

        @pl.when(c == 1)
        def _():
            work(hb_h, out1_h)

    return k(ha, hb, src2, dst2)


BLK = 1024


def _dinv_block(c0_ref, c1_ref):
    deg = c0_ref[:, 0:1] + c1_ref[:, 0:1] + 1.0
    return lax.rsqrt(deg)


def _mm_xw(x, w):
    d_out = w.shape[1]
    dh = d_out // 2

    def body(x_ref, w_ref, ol_ref, or_ref):
        h = jnp.dot(x_ref[...], w_ref[...],
                    preferred_element_type=jnp.float32)
        ol_ref[...] = h[:, :dh]
        or_ref[...] = h[:, dh:]

    return pl.pallas_call(
        body,
        grid=(N_PAD // BLK,),
        in_specs=[
            pl.BlockSpec((BLK, x.shape[1]), lambda i: (i, 0)),
            pl.BlockSpec(w.shape, lambda i: (0, 0)),
        ],
        out_specs=[
            pl.BlockSpec((BLK, dh), lambda i: (i, 0)),
            pl.BlockSpec((BLK, dh), lambda i: (i, 0)),
        ],
        out_shape=[
            jax.ShapeDtypeStruct((N_PAD, dh), jnp.float32),
            jax.ShapeDtypeStruct((N_PAD, dh), jnp.float32),
        ],
    )(x, w)


def _scale_halves(xwl, xwr, cnt0, cnt1):
    dh = xwl.shape[1]

    def body(xl_ref, xr_ref, c0_ref, c1_ref, ol_ref, or_ref):
        dinv = _dinv_block(c0_ref, c1_ref)
        ol_ref[...] = xl_ref[...] * dinv
        or_ref[...] = xr_ref[...] * dinv

    return pl.pallas_call(
        body,
        grid=(N_PAD // BLK,),
        in_specs=[
            pl.BlockSpec((BLK, dh), lambda i: (i, 0)),
            pl.BlockSpec((BLK, dh), lambda i: (i, 0)),
            pl.BlockSpec((BLK, DEG_W), lambda i: (i, 0)),
            pl.BlockSpec((BLK, DEG_W), lambda i: (i, 0)),
        ],
        out_specs=[
            pl.BlockSpec((BLK, dh), lambda i: (i, 0)),
            pl.BlockSpec((BLK, dh), lambda i: (i, 0)),
        ],
        out_shape=[
            jax.ShapeDtypeStruct((N_PAD, dh), jnp.float32),
            jax.ShapeDtypeStruct((N_PAD, dh), jnp.float32),
        ],
    )(xwl, xwr, cnt0, cnt1)


def _mm_mid(accl, accr, cnt0, cnt1, b_row, w):
    d_in = accl.shape[1] + accr.shape[1]
    dh_in = accl.shape[1]
    d_out = w.shape[1]
    dh = d_out // 2

    def body(al_ref, ar_ref, c0_ref, c1_ref, b_ref, w_ref, ol_ref, or_ref):
        dinv = _dinv_block(c0_ref, c1_ref)
        a = jnp.concatenate([al_ref[...], ar_ref[...]], axis=1)
        xr = jnp.maximum(a * dinv + b_ref[...], 0.0)
        h = jnp.dot(xr, w_ref[...], preferred_element_type=jnp.float32) * dinv
        ol_ref[...] = h[:, :dh]
        or_ref[...] = h[:, dh:]

    return pl.pallas_call(
        body,
        grid=(N_PAD // BLK,),
        in_specs=[
            pl.BlockSpec((BLK, dh_in), lambda i: (i, 0)),
            pl.BlockSpec((BLK, dh_in), lambda i: (i, 0)),
            pl.BlockSpec((BLK, DEG_W), lambda i: (i, 0)),
            pl.BlockSpec((BLK, DEG_W), lambda i: (i, 0)),
            pl.BlockSpec((1, d_in), lambda i: (0, 0)),
            pl.BlockSpec(w.shape, lambda i: (0, 0)),
        ],
        out_specs=[
            pl.BlockSpec((BLK, dh), lambda i: (i, 0)),
            pl.BlockSpec((BLK, dh), lambda i: (i, 0)),
        ],
        out_shape=[
            jax.ShapeDtypeStruct((N_PAD, dh), jnp.float32),
            jax.ShapeDtypeStruct((N_PAD, dh), jnp.float32),
        ],
    )(accl, accr, cnt0, cnt1, b_row, w)


def _mm_last(accl, accr, cnt0, cnt1, b_row, w_pad):
    d_in = accl.shape[1] + accr.shape[1]
    dh_in = accl.shape[1]
    d_out = w_pad.shape[1]

    def body(al_ref, ar_ref, c0_ref, c1_ref, b_ref, w_ref, oa_ref, ob_ref):
        dinv = _dinv_block(c0_ref, c1_ref)
        a = jnp.concatenate([al_ref[...], ar_ref[...]], axis=1)
        xr = jnp.maximum(a * dinv + b_ref[...], 0.0)
        h = jnp.dot(xr, w_ref[...],
                    preferred_element_type=jnp.float32) * dinv
        oa_ref[...] = h
        ob_ref[...] = h

    return pl.pallas_call(
        body,
        grid=(N_PAD // BLK,),
        in_specs=[
            pl.BlockSpec((BLK, dh_in), lambda i: (i, 0)),
            pl.BlockSpec((BLK, dh_in), lambda i: (i, 0)),
            pl.BlockSpec((BLK, DEG_W), lambda i: (i, 0)),
            pl.BlockSpec((BLK, DEG_W), lambda i: (i, 0)),
            pl.BlockSpec((1, d_in), lambda i: (0, 0)),
            pl.BlockSpec(w_pad.shape, lambda i: (0, 0)),
        ],
        out_specs=[
            pl.BlockSpec((BLK, d_out), lambda i: (i, 0)),
            pl.BlockSpec((BLK, d_out), lambda i: (i, 0)),
        ],
        out_shape=[
            jax.ShapeDtypeStruct((N_PAD, d_out), jnp.float32),
            jax.ShapeDtypeStruct((N_PAD, d_out), jnp.float32),
        ],
    )(accl, accr, cnt0, cnt1, b_row, w_pad)


def _final(p0, p1, h3, cnt0, cnt1, b_row, n_cls):
    def body(p0_ref, p1_ref, h3_ref, c0_ref, c1_ref, b_ref, o_ref):
        dinv = _dinv_block(c0_ref, c1_ref)
        acc = p0_ref[:, :n_cls] + p1_ref[:, :n_cls] - h3_ref[:, :n_cls]
        logits = acc * dinv + b_ref[...]
        m = jnp.max(logits, axis=1, keepdims=True)
        e = jnp.exp(logits - m)
        lse = jnp.log(jnp.sum(e, axis=1, keepdims=True)) + m
        o_ref[...] = logits - lse

    return pl.pallas_call(
        body,
        grid=(N_PAD // BLK,),
        in_specs=[
            pl.BlockSpec((BLK, 128), lambda i: (i, 0)),
            pl.BlockSpec((BLK, 128), lambda i: (i, 0)),
            pl.BlockSpec((BLK, 128), lambda i: (i, 0)),
            pl.BlockSpec((BLK, DEG_W), lambda i: (i, 0)),
            pl.BlockSpec((BLK, DEG_W), lambda i: (i, 0)),
            pl.BlockSpec((1, n_cls), lambda i: (0, 0)),
        ],
        out_specs=pl.BlockSpec((BLK, n_cls), lambda i: (i, 0)),
        out_shape=jax.ShapeDtypeStruct((N_PAD, n_cls), jnp.float32),
    )(p0, p1, h3, cnt0, cnt1, b_row)


def kernel(x, edge_index, W1, b1, W2, b2, W3, b3):
    src = jnp.pad(edge_index[0], (0, E_PAD - E), constant_values=PAD_ROW)
    dst = jnp.pad(edge_index[1], (0, E_PAD - E), constant_values=PAD_ROW)
    src3 = src.reshape(NT * CHT, CHUNK)
    dst3 = dst.reshape(NT * CHT, CHUNK)
    dst3_deg = dst.reshape(32, DEG_CHT, CHUNK)
    x_pad = jnp.pad(x, ((0, N_PAD - N), (0, 0)))
    ones_rows = jnp.ones((CHUNK, DEG_W), jnp.float32)
    zero_rows = jnp.zeros((ZROWS, DEG_W), jnp.float32)

    xwl, xwr = _mm_xw(x_pad, W1)
    cnt0, cnt1 = _degree_counts(dst3_deg, ones_rows, zero_rows)

    hl1, hr1 = _scale_halves(xwl, xwr, cnt0, cnt1)
    acc1l, acc1r = _propagate(hl1, hr1, src3, dst3, 128)

    hl2, hr2 = _mm_mid(acc1l, acc1r, cnt0, cnt1, b1.reshape(1, -1), W2)
    acc2l, acc2r = _propagate(hl2, hr2, src3, dst3, 128)

    n_cls = W3.shape[1]
    w3_pad = jnp.pad(W3, ((0, 0), (0, 128 - n_cls)))
    h3a, h3b = _mm_last(acc2l, acc2r, cnt0, cnt1, b2.reshape(1, -1), w3_pad)
    p0, p1 = _propagate_last(h3a, h3b, src3, dst3)

    out = _final(p0, p1, h3a, cnt0, cnt1, b3.reshape(1, -1), n_cls)
    return out[:N]
